# baseline (speedup 1.0000x reference)
import functools

import jax
import jax.numpy as jnp
from jax import lax
from jax.experimental import pallas as pl
from jax.experimental.pallas import tpu as pltpu
from jax.experimental.pallas import tpu_sc as plsc

_LANES = 16
_NUM_DET = 100
_SEL0 = 100
_BASE = 96
_NWORK = 7
_ROWS = 84
_W = _NWORK * _LANES
_COLS = 256


def _splat(cmv, k):
    return cmv[pl.ds(k * _LANES, _LANES)]


@functools.partial(
    pl.kernel,
    out_type=jax.ShapeDtypeStruct((_W, 8), jnp.float32),
    mesh=plsc.VectorSubcoreMesh(
        core_axis_name="c", subcore_axis_name="s", num_cores=1),
    scratch_types=[
        pltpu.VMEM((_ROWS, _COLS), jnp.float32),
        pltpu.VMEM((16 * _LANES,), jnp.float32),
        pltpu.VMEM((_LANES, 8), jnp.float32),
        pltpu.SemaphoreType.DMA,
        pltpu.SemaphoreType.DMA,
    ],
    compiler_params=pltpu.CompilerParams(
        needs_layout_passes=False,
        skip_device_barrier=True,
    ),
)
def _sc_detect(x_hbm, cm_hbm, out_hbm, xv, cmv, outv, sem1, sem2):
    wid = lax.axis_index("s")

    @pl.when(wid < _NWORK)
    def _():
        cp1 = pltpu.async_copy(x_hbm.at[:, pl.ds(0, _COLS)], xv, sem1)
        cp2 = pltpu.async_copy(cm_hbm, cmv, sem2)
        cp1.wait()
        cp2.wait()
        col = _BASE + wid * _LANES

        def step(c, carry):
            best, best_id = carry
            s = xv[4 + c, pl.ds(col, _LANES)]
            pr = s > best
            cf = c.astype(jnp.float32)
            return (jnp.where(pr, s, best),
                    jnp.where(pr, jnp.broadcast_to(cf, (_LANES,)), best_id))

        best, best_id = lax.fori_loop(
            1, _ROWS - 4, step,
            (xv[4, pl.ds(col, _LANES)],
             jnp.zeros((_LANES,), jnp.float32)))

        b = tuple(xv[i, pl.ds(col, _LANES)] for i in range(4))
        lanes = lax.iota(jnp.int32, _LANES)
        zeros = jnp.zeros((_LANES,), jnp.float32)
        plsc.store_scatter(outv, [lanes, zeros.astype(jnp.int32)], zeros)
        for j in range(4):
            acc = b[0] * _splat(cmv, j)
            for i in range(1, 4):
                acc = acc + b[i] * _splat(cmv, i * 4 + j)
            plsc.store_scatter(
                outv, [lanes, jnp.full((_LANES,), 1 + j, jnp.int32)], acc)
        plsc.store_scatter(outv, [lanes, jnp.full((_LANES,), 5, jnp.int32)],
                           best_id)
        plsc.store_scatter(outv, [lanes, jnp.full((_LANES,), 6, jnp.int32)],
                           best)
        plsc.store_scatter(outv, [lanes, jnp.full((_LANES,), 7, jnp.int32)],
                           zeros)

        pltpu.sync_copy(outv, out_hbm.at[pl.ds(wid * _LANES, _LANES), :])


def kernel(x, convert_matrix):
    x2 = x.reshape(x.shape[1], x.shape[2])
    cmb = jnp.tile(convert_matrix.reshape(16, 1), (1, _LANES)).reshape(-1)
    staged = _sc_detect(x2, cmb)
    off = _SEL0 - _BASE
    return staged[off:off + _NUM_DET, :7]


# speedup vs baseline: 1.0344x; 1.0344x over previous
import functools

import jax
import jax.numpy as jnp
from jax import lax
from jax.experimental import pallas as pl
from jax.experimental.pallas import tpu as pltpu
from jax.experimental.pallas import tpu_sc as plsc

_LANES = 16
_NUM_DET = 100
_SEL0 = 100
_BASE = 96
_NWORK = 7
_ROWS = 84
_W = _NWORK * _LANES
_COLS = 256


def _splat(cmv, k):
    return cmv[pl.ds(k * _LANES, _LANES)]


@functools.partial(
    pl.kernel,
    out_type=jax.ShapeDtypeStruct((_W, 8), jnp.float32),
    mesh=plsc.VectorSubcoreMesh(
        core_axis_name="c", subcore_axis_name="s", num_cores=1),
    scratch_types=[
        pltpu.VMEM((_ROWS, 128), jnp.float32),
        pltpu.VMEM((16 * _LANES,), jnp.float32),
        pltpu.VMEM((_LANES, 8), jnp.float32),
        pltpu.SemaphoreType.DMA,
        pltpu.SemaphoreType.DMA,
    ],
    compiler_params=pltpu.CompilerParams(
        needs_layout_passes=False,
        skip_device_barrier=True,
    ),
)
def _sc_detect(x_hbm, cm_hbm, out_hbm, xv, cmv, outv, sem1, sem2):
    wid = lax.axis_index("s")

    @pl.when(wid < _NWORK)
    def _():
        tile = (_BASE + wid * _LANES) // 128
        cp1 = pltpu.async_copy(
            x_hbm.at[:, pl.ds(pl.multiple_of(tile * 128, 128), 128)], xv, sem1)
        cp2 = pltpu.async_copy(cm_hbm, cmv, sem2)
        cp1.wait()
        cp2.wait()
        col = _BASE + wid * _LANES - tile * 128

        def step(c, carry):
            best, best_id = carry
            s = xv[4 + c, pl.ds(col, _LANES)]
            pr = s > best
            cf = c.astype(jnp.float32)
            return (jnp.where(pr, s, best),
                    jnp.where(pr, jnp.broadcast_to(cf, (_LANES,)), best_id))

        best, best_id = lax.fori_loop(
            1, _ROWS - 4, step,
            (xv[4, pl.ds(col, _LANES)],
             jnp.zeros((_LANES,), jnp.float32)))

        b = tuple(xv[i, pl.ds(col, _LANES)] for i in range(4))
        lanes = lax.iota(jnp.int32, _LANES)
        zeros = jnp.zeros((_LANES,), jnp.float32)
        plsc.store_scatter(outv, [lanes, zeros.astype(jnp.int32)], zeros)
        for j in range(4):
            acc = b[0] * _splat(cmv, j)
            for i in range(1, 4):
                acc = acc + b[i] * _splat(cmv, i * 4 + j)
            plsc.store_scatter(
                outv, [lanes, jnp.full((_LANES,), 1 + j, jnp.int32)], acc)
        plsc.store_scatter(outv, [lanes, jnp.full((_LANES,), 5, jnp.int32)],
                           best_id)
        plsc.store_scatter(outv, [lanes, jnp.full((_LANES,), 6, jnp.int32)],
                           best)
        plsc.store_scatter(outv, [lanes, jnp.full((_LANES,), 7, jnp.int32)],
                           zeros)

        pltpu.sync_copy(outv, out_hbm.at[pl.ds(wid * _LANES, _LANES), :])


def kernel(x, convert_matrix):
    x2 = x.reshape(x.shape[1], x.shape[2])
    cmb = jnp.tile(convert_matrix.reshape(16, 1), (1, _LANES)).reshape(-1)
    staged = _sc_detect(x2, cmb)
    off = _SEL0 - _BASE
    return staged[off:off + _NUM_DET, :7]


# speedup vs baseline: 1.0758x; 1.0401x over previous
import functools

import jax
import jax.numpy as jnp
from jax import lax
from jax.experimental import pallas as pl
from jax.experimental.pallas import tpu as pltpu
from jax.experimental.pallas import tpu_sc as plsc

_LANES = 16
_NUM_DET = 100
_SEL0 = 100
_BASE = 96
_NWORK = 7
_ROWS = 84
_W = _NWORK * _LANES
_COLS = 256


@functools.partial(
    pl.kernel,
    out_type=jax.ShapeDtypeStruct((_W, 8), jnp.float32),
    mesh=plsc.VectorSubcoreMesh(
        core_axis_name="c", subcore_axis_name="s", num_cores=1),
    scratch_types=[
        pltpu.VMEM((_ROWS, 128), jnp.float32),
        pltpu.VMEM((_LANES,), jnp.float32),
        pltpu.VMEM((_LANES, 8), jnp.float32),
        pltpu.SemaphoreType.DMA,
        pltpu.SemaphoreType.DMA,
    ],
    compiler_params=pltpu.CompilerParams(
        needs_layout_passes=False,
        skip_device_barrier=True,
    ),
)
def _sc_detect(x_hbm, cm_hbm, out_hbm, xv, cmv, outv, sem1, sem2):
    wid = lax.axis_index("s")

    @pl.when(wid < _NWORK)
    def _():
        tile = (_BASE + wid * _LANES) // 128
        cp1 = pltpu.async_copy(
            x_hbm.at[:, pl.ds(pl.multiple_of(tile * 128, 128), 128)], xv, sem1)
        cp2 = pltpu.async_copy(cm_hbm, cmv, sem2)
        cp1.wait()
        cp2.wait()
        col = _BASE + wid * _LANES - tile * 128

        def step(c, carry):
            best, best_id = carry
            s = xv[4 + c, pl.ds(col, _LANES)]
            pr = s > best
            cf = c.astype(jnp.float32)
            return (jnp.where(pr, s, best),
                    jnp.where(pr, jnp.broadcast_to(cf, (_LANES,)), best_id))

        best, best_id = lax.fori_loop(
            1, _ROWS - 4, step,
            (xv[4, pl.ds(col, _LANES)],
             jnp.zeros((_LANES,), jnp.float32)),
            unroll=8)

        b = tuple(xv[i, pl.ds(col, _LANES)] for i in range(4))
        cmvec = cmv[...]
        lanes = lax.iota(jnp.int32, _LANES)
        zeros = jnp.zeros((_LANES,), jnp.float32)
        plsc.store_scatter(outv, [lanes, zeros.astype(jnp.int32)], zeros)
        for j in range(4):
            acc = b[0] * cmvec[j]
            for i in range(1, 4):
                acc = acc + b[i] * cmvec[i * 4 + j]
            plsc.store_scatter(
                outv, [lanes, jnp.full((_LANES,), 1 + j, jnp.int32)], acc)
        plsc.store_scatter(outv, [lanes, jnp.full((_LANES,), 5, jnp.int32)],
                           best_id)
        plsc.store_scatter(outv, [lanes, jnp.full((_LANES,), 6, jnp.int32)],
                           best)
        plsc.store_scatter(outv, [lanes, jnp.full((_LANES,), 7, jnp.int32)],
                           zeros)

        pltpu.sync_copy(outv, out_hbm.at[pl.ds(wid * _LANES, _LANES), :])


def kernel(x, convert_matrix):
    xs = x.reshape(x.shape[1], x.shape[2])[:, :_COLS]
    staged = _sc_detect(xs, convert_matrix.reshape(16))
    off = _SEL0 - _BASE
    return staged[off:off + _NUM_DET, :7]
